# baseline (speedup 1.0000x reference)
import functools

import jax
import jax.numpy as jnp
from jax import lax
from jax.experimental import pallas as pl
from jax.experimental.pallas import tpu as pltpu
from jax.experimental.pallas import tpu_sc as plsc

N = 65536
C = 80
TOTAL = N * C
NW = 32
PER_W = TOTAL // NW
CHUNK = 128 * C
NCH = PER_W // CHUNK
ROWS = CHUNK // C
L = 16
VPR = C // L


def _log1p01(u):
    z = u / (2.0 + u)
    z2 = z * z
    p = jnp.float32(1.0 / 9.0)
    for c in (1.0 / 7.0, 1.0 / 5.0, 1.0 / 3.0, 1.0):
        p = p * z2 + jnp.float32(c)
    return 2.0 * z * p


def _pow15(a):
    i = lax.bitcast_convert_type(a, jnp.int32)
    y = lax.bitcast_convert_type(
        jnp.int32(0x5F3759DF) - lax.shift_right_arithmetic(i, 1), jnp.float32)
    y = y * (1.5 - 0.5 * a * y * y)
    y = y * (1.5 - 0.5 * a * y * y)
    return a * a * y


def _elem(x, lv, sc):
    ax = jnp.abs(x)
    u = jnp.exp(-ax)
    sp = jnp.maximum(x, 0.0) + _log1p01(u)
    d = 1.0 / (1.0 + u)
    s = jnp.where(x >= 0.0, d, 1.0 - d)
    neg = sp * _pow15(s)
    pos = (sp - x * sc) * _pow15(jnp.abs(sc - s))
    return jnp.where(lv > 0, pos, neg)


def _qfl_body(pred_h, lab_h, score_h, out_h,
              pb0, pb1, lb0, lb1, sc_v, acc_v,
              sp0, sp1, sl0, sl1):
    wid = lax.axis_index("s") * 2 + lax.axis_index("c")
    base = wid * PER_W
    last = base + (NCH - 1) * CHUNK

    pltpu.sync_copy(score_h, sc_v)
    scv = [sc_v[pl.ds(L * v, L)] for v in range(VPR)]

    def start(c_off, pb, lb, sp, sl):
        off = jnp.minimum(c_off, last)
        pltpu.async_copy(pred_h.at[pl.ds(off, CHUNK)], pb, sp)
        pltpu.async_copy(lab_h.at[pl.ds(off, CHUNK)], lb, sl)

    def wait(pb, lb, sp, sl):
        pltpu.make_async_copy(pred_h.at[pl.ds(base, CHUNK)], pb, sp).wait()
        pltpu.make_async_copy(lab_h.at[pl.ds(base, CHUNK)], lb, sl).wait()

    def compute(pb, lb, acc):
        def row(r, acc):
            b = pl.multiple_of(r * C, L)
            for v in range(VPR):
                x = pb[pl.ds(b + L * v, L)]
                lv = lb[pl.ds(b + L * v, L)]
                acc = acc + _elem(x, lv, scv[v])
            return acc
        return lax.fori_loop(0, ROWS, row, acc)

    start(base, pb0, lb0, sp0, sl0)
    start(base + CHUNK, pb1, lb1, sp1, sl1)

    def pair(g, acc):
        c0 = base + (2 * g) * CHUNK
        wait(pb0, lb0, sp0, sl0)
        acc = compute(pb0, lb0, acc)
        start(c0 + 2 * CHUNK, pb0, lb0, sp0, sl0)
        wait(pb1, lb1, sp1, sl1)
        acc = compute(pb1, lb1, acc)
        start(c0 + 3 * CHUNK, pb1, lb1, sp1, sl1)
        return acc

    acc = lax.fori_loop(0, NCH // 2, pair, jnp.zeros((L,), jnp.float32))

    wait(pb0, lb0, sp0, sl0)
    wait(pb1, lb1, sp1, sl1)

    acc_v[...] = acc
    pltpu.sync_copy(acc_v, out_h.at[wid])


@jax.jit
def kernel(pred, label, score):
    mesh = plsc.VectorSubcoreMesh(core_axis_name="c", subcore_axis_name="s")
    f = functools.partial(
        pl.kernel,
        mesh=mesh,
        out_type=jax.ShapeDtypeStruct((NW, L), jnp.float32),
        scratch_types=[
            pltpu.VMEM((CHUNK,), jnp.float32),
            pltpu.VMEM((CHUNK,), jnp.float32),
            pltpu.VMEM((CHUNK,), jnp.int32),
            pltpu.VMEM((CHUNK,), jnp.int32),
            pltpu.VMEM((C,), jnp.float32),
            pltpu.VMEM((L,), jnp.float32),
            pltpu.SemaphoreType.DMA,
            pltpu.SemaphoreType.DMA,
            pltpu.SemaphoreType.DMA,
            pltpu.SemaphoreType.DMA,
        ],
    )(_qfl_body)
    partials = f(pred.reshape(TOTAL), label.reshape(TOTAL), score)
    return jnp.sum(partials) / jnp.float32(TOTAL)


# speedup vs baseline: 1.1755x; 1.1755x over previous
import functools

import jax
import jax.numpy as jnp
from jax import lax
from jax.experimental import pallas as pl
from jax.experimental.pallas import tpu as pltpu
from jax.experimental.pallas import tpu_sc as plsc

N = 65536
C = 80
TOTAL = N * C
NW = 32
PER_W = TOTAL // NW
CHUNK = 128 * C
NCH = PER_W // CHUNK
ROWS = CHUNK // C
L = 16
VPR = C // L


_LOG1P_C = (1.6936626598407223e-06, 0.9998325947816316, -0.49720333122019134,
            0.31504127990864345, -0.18901954822291905, 0.08152317761736225,
            -0.017029610589052675)


def _log1p01(u):
    p = jnp.float32(_LOG1P_C[6])
    for c in _LOG1P_C[5::-1]:
        p = p * u + jnp.float32(c)
    return p


def _pow15(a):
    i = lax.bitcast_convert_type(a, jnp.int32)
    y = lax.bitcast_convert_type(
        jnp.int32(0x5F3759DF) - lax.shift_right_arithmetic(i, 1), jnp.float32)
    y = y * (1.5 - 0.5 * a * y * y)
    y = y * (1.5 - 0.5 * a * y * y)
    return a * a * y


def _elem(x, lv, sc):
    ax = jnp.abs(x)
    u = jnp.exp(-ax)
    d = 1.0 / (1.0 + u)
    sp = jnp.maximum(x, 0.0) + _log1p01(u)
    s = jnp.where(x >= 0.0, d, 1.0 - d)
    pos = lv > 0
    scm = jnp.where(pos, sc, 0.0)
    a = jnp.where(pos, jnp.abs(sc - s), s)
    return (sp - x * scm) * _pow15(a)


def _qfl_body(pred_h, lab_h, score_h, out_h,
              pb0, pb1, lb0, lb1, sc_v, acc_v,
              sp0, sp1, sl0, sl1):
    wid = lax.axis_index("s") * 2 + lax.axis_index("c")
    base = wid * PER_W
    last = base + (NCH - 1) * CHUNK

    pltpu.sync_copy(score_h, sc_v)
    scv = [sc_v[pl.ds(L * v, L)] for v in range(VPR)]

    def start(c_off, pb, lb, sp, sl):
        off = jnp.minimum(c_off, last)
        pltpu.async_copy(pred_h.at[pl.ds(off, CHUNK)], pb, sp)
        pltpu.async_copy(lab_h.at[pl.ds(off, CHUNK)], lb, sl)

    def wait(pb, lb, sp, sl):
        pltpu.make_async_copy(pred_h.at[pl.ds(base, CHUNK)], pb, sp).wait()
        pltpu.make_async_copy(lab_h.at[pl.ds(base, CHUNK)], lb, sl).wait()

    def compute(pb, lb, acc):
        def row(r, acc):
            b = pl.multiple_of(r * C, L)
            for v in range(VPR):
                x = pb[pl.ds(b + L * v, L)]
                lv = lb[pl.ds(b + L * v, L)]
                acc = acc + _elem(x, lv, scv[v])
            return acc
        return lax.fori_loop(0, ROWS, row, acc)

    start(base, pb0, lb0, sp0, sl0)
    start(base + CHUNK, pb1, lb1, sp1, sl1)

    def pair(g, acc):
        c0 = base + (2 * g) * CHUNK
        wait(pb0, lb0, sp0, sl0)
        acc = compute(pb0, lb0, acc)
        start(c0 + 2 * CHUNK, pb0, lb0, sp0, sl0)
        wait(pb1, lb1, sp1, sl1)
        acc = compute(pb1, lb1, acc)
        start(c0 + 3 * CHUNK, pb1, lb1, sp1, sl1)
        return acc

    acc = lax.fori_loop(0, NCH // 2, pair, jnp.zeros((L,), jnp.float32))

    wait(pb0, lb0, sp0, sl0)
    wait(pb1, lb1, sp1, sl1)

    acc_v[...] = acc
    pltpu.sync_copy(acc_v, out_h.at[wid])


@jax.jit
def kernel(pred, label, score):
    mesh = plsc.VectorSubcoreMesh(core_axis_name="c", subcore_axis_name="s")
    f = functools.partial(
        pl.kernel,
        mesh=mesh,
        out_type=jax.ShapeDtypeStruct((NW, L), jnp.float32),
        scratch_types=[
            pltpu.VMEM((CHUNK,), jnp.float32),
            pltpu.VMEM((CHUNK,), jnp.float32),
            pltpu.VMEM((CHUNK,), jnp.int32),
            pltpu.VMEM((CHUNK,), jnp.int32),
            pltpu.VMEM((C,), jnp.float32),
            pltpu.VMEM((L,), jnp.float32),
            pltpu.SemaphoreType.DMA,
            pltpu.SemaphoreType.DMA,
            pltpu.SemaphoreType.DMA,
            pltpu.SemaphoreType.DMA,
        ],
    )(_qfl_body)
    partials = f(pred.reshape(TOTAL), label.reshape(TOTAL), score)
    return jnp.sum(partials) / jnp.float32(TOTAL)


# speedup vs baseline: 1.4674x; 1.2483x over previous
import functools

import jax
import jax.numpy as jnp
from jax import lax
from jax.experimental import pallas as pl
from jax.experimental.pallas import tpu as pltpu
from jax.experimental.pallas import tpu_sc as plsc

N = 65536
C = 80
TOTAL = N * C
L = 16
VPR = C // L

R_SC = 12288
R_TC = N - R_SC
NW = 32
PER_W = R_SC * C // NW
CHUNK = 128 * C
NCH = PER_W // CHUNK
ROWS = CHUNK // C

BR = 4096
G_TC = R_TC // BR

_LOG1P_C = (1.6936626598407223e-06, 0.9998325947816316, -0.49720333122019134,
            0.31504127990864345, -0.18901954822291905, 0.08152317761736225,
            -0.017029610589052675)


def _log1p01(u):
    p = jnp.float32(_LOG1P_C[6])
    for c in _LOG1P_C[5::-1]:
        p = p * u + jnp.float32(c)
    return p


def _pow15(a):
    i = lax.bitcast_convert_type(a, jnp.int32)
    y = lax.bitcast_convert_type(
        jnp.int32(0x5F3759DF) - lax.shift_right_arithmetic(i, 1), jnp.float32)
    y = y * (1.5 - 0.5 * a * y * y)
    y = y * (1.5 - 0.5 * a * y * y)
    return a * a * y


def _elem(x, pos, sc):
    ax = jnp.abs(x)
    u = jnp.exp(-ax)
    d = 1.0 / (1.0 + u)
    sp = jnp.maximum(x, 0.0) + _log1p01(u)
    s = jnp.where(x >= 0.0, d, 1.0 - d)
    scm = jnp.where(pos, sc, 0.0)
    a = jnp.where(pos, jnp.abs(sc - s), s)
    return (sp - x * scm) * _pow15(a)


def _sc_body(pred_h, lab_h, score_h, out_h,
             pb0, pb1, lb0, lb1, sc_v, acc_v,
             sp0, sp1, sl0, sl1):
    wid = lax.axis_index("s") * 2 + lax.axis_index("c")
    base = wid * PER_W
    last = base + (NCH - 1) * CHUNK

    pltpu.sync_copy(score_h, sc_v)
    scv = [sc_v[pl.ds(L * v, L)] for v in range(VPR)]

    def start(c_off, pb, lb, sp, sl):
        off = jnp.minimum(c_off, last)
        pltpu.async_copy(pred_h.at[pl.ds(off, CHUNK)], pb, sp)
        pltpu.async_copy(lab_h.at[pl.ds(off, CHUNK)], lb, sl)

    def wait(pb, lb, sp, sl):
        pltpu.make_async_copy(pred_h.at[pl.ds(base, CHUNK)], pb, sp).wait()
        pltpu.make_async_copy(lab_h.at[pl.ds(base, CHUNK)], lb, sl).wait()

    def compute(pb, lb, acc):
        def row(r, acc):
            b = pl.multiple_of(r * C, L)
            for v in range(VPR):
                x = pb[pl.ds(b + L * v, L)]
                lv = lb[pl.ds(b + L * v, L)]
                acc = acc + _elem(x, lv > 0.0, scv[v])
            return acc
        return lax.fori_loop(0, ROWS, row, acc)

    start(base, pb0, lb0, sp0, sl0)
    start(base + CHUNK, pb1, lb1, sp1, sl1)

    def pair(g, acc):
        c0 = base + (2 * g) * CHUNK
        wait(pb0, lb0, sp0, sl0)
        acc = compute(pb0, lb0, acc)
        start(c0 + 2 * CHUNK, pb0, lb0, sp0, sl0)
        wait(pb1, lb1, sp1, sl1)
        acc = compute(pb1, lb1, acc)
        start(c0 + 3 * CHUNK, pb1, lb1, sp1, sl1)
        return acc

    acc = lax.fori_loop(0, NCH // 2, pair, jnp.zeros((L,), jnp.float32))

    if NCH % 2:
        wait(pb0, lb0, sp0, sl0)
        acc = compute(pb0, lb0, acc)
        start(last, pb0, lb0, sp0, sl0)

    wait(pb0, lb0, sp0, sl0)
    wait(pb1, lb1, sp1, sl1)

    acc_v[...] = acc
    pltpu.sync_copy(acc_v, out_h.at[wid])


def _sc_call(pred_flat, lab_flat, score):
    mesh = plsc.VectorSubcoreMesh(core_axis_name="c", subcore_axis_name="s")
    f = functools.partial(
        pl.kernel,
        mesh=mesh,
        out_type=jax.ShapeDtypeStruct((NW, L), jnp.float32),
        scratch_types=[
            pltpu.VMEM((CHUNK,), jnp.float32),
            pltpu.VMEM((CHUNK,), jnp.float32),
            pltpu.VMEM((CHUNK,), jnp.float32),
            pltpu.VMEM((CHUNK,), jnp.float32),
            pltpu.VMEM((C,), jnp.float32),
            pltpu.VMEM((L,), jnp.float32),
            pltpu.SemaphoreType.DMA,
            pltpu.SemaphoreType.DMA,
            pltpu.SemaphoreType.DMA,
            pltpu.SemaphoreType.DMA,
        ],
    )(_sc_body)
    return f(pred_flat, lab_flat, score)


def _tc_body(score_ref, pred_ref, lab_ref, out_ref):
    x = pred_ref[...]
    pos = lab_ref[...] > 0
    sc = score_ref[...]
    ax = jnp.abs(x)
    u = jnp.exp(-ax)
    d = 1.0 / (1.0 + u)
    sp = jnp.maximum(x, 0.0) + jnp.log1p(u)
    s = jnp.where(x >= 0.0, d, 1.0 - d)
    scm = jnp.where(pos, sc, 0.0)
    a = jnp.where(pos, jnp.abs(sc - s), s)
    res = (sp - x * scm) * (a * lax.sqrt(a))

    @pl.when(pl.program_id(0) == 0)
    def _():
        out_ref[0, 0] = 0.0

    out_ref[0, 0] += jnp.sum(res)


def _tc_call(pred, label, score):
    return pl.pallas_call(
        _tc_body,
        grid=(G_TC,),
        in_specs=[
            pl.BlockSpec((1, C), lambda i: (0, 0)),
            pl.BlockSpec((BR, C), lambda i: (i, 0)),
            pl.BlockSpec((BR, C), lambda i: (i, 0)),
        ],
        out_specs=pl.BlockSpec((1, 1), lambda i: (0, 0),
                               memory_space=pltpu.SMEM),
        out_shape=jax.ShapeDtypeStruct((1, 1), jnp.float32),
    )(score.reshape(1, C), pred, label)


@jax.jit
def kernel(pred, label, score):
    tc_part = _tc_call(pred[:R_TC], label[:R_TC], score)
    pred_sc = pred[R_TC:].reshape(R_SC * C)
    lab_sc = label[R_TC:].astype(jnp.float32).reshape(R_SC * C)
    sc_part = _sc_call(pred_sc, lab_sc, score)
    return (jnp.sum(tc_part) + jnp.sum(sc_part)) / jnp.float32(TOTAL)


# speedup vs baseline: 2.0355x; 1.3871x over previous
import functools

import jax
import jax.numpy as jnp
from jax import lax
from jax.experimental import pallas as pl
from jax.experimental.pallas import tpu as pltpu
from jax.experimental.pallas import tpu_sc as plsc

N = 65536
C = 80
TOTAL = N * C
L = 16
VPR = C // L

R_SC = 12288
R_TC = N - R_SC
NW = 32
RPW = R_SC // NW
CROWS = 128
NCH = RPW // CROWS

BR = 4096
G_TC = R_TC // BR

_LOG1P_C = (1.6936626598407223e-06, 0.9998325947816316, -0.49720333122019134,
            0.31504127990864345, -0.18901954822291905, 0.08152317761736225,
            -0.017029610589052675)


def _log1p01(u):
    p = jnp.float32(_LOG1P_C[6])
    for c in _LOG1P_C[5::-1]:
        p = p * u + jnp.float32(c)
    return p


def _pow15(a):
    i = lax.bitcast_convert_type(a, jnp.int32)
    y = lax.bitcast_convert_type(
        jnp.int32(0x5F3759DF) - lax.shift_right_arithmetic(i, 1), jnp.float32)
    y = y * (1.5 - 0.5 * a * y * y)
    y = y * (1.5 - 0.5 * a * y * y)
    return a * a * y


def _elem(x, pos, sc):
    ax = jnp.abs(x)
    u = jnp.exp(-ax)
    d = 1.0 / (1.0 + u)
    sp = jnp.maximum(x, 0.0) + _log1p01(u)
    s = jnp.where(x >= 0.0, d, 1.0 - d)
    scm = jnp.where(pos, sc, 0.0)
    a = jnp.where(pos, jnp.abs(sc - s), s)
    return (sp - x * scm) * _pow15(a)


def _sc_body(pred_h, lab_h, score_h, out_h,
             pb0, pb1, lb0, lb1, sc_v, acc_v,
             sp0, sp1, sl0, sl1):
    wid = lax.axis_index("s") * 2 + lax.axis_index("c")
    base = R_TC + wid * RPW
    last = base + (NCH - 1) * CROWS

    pltpu.sync_copy(score_h, sc_v)
    scv = [sc_v[pl.ds(L * v, L)] for v in range(VPR)]

    def start(row0, pb, lb, sp, sl):
        row = jnp.minimum(row0, last)
        pltpu.async_copy(pred_h.at[pl.ds(row, CROWS)], pb, sp)
        pltpu.async_copy(lab_h.at[pl.ds(row, CROWS)], lb, sl)

    def wait(pb, lb, sp, sl):
        pltpu.make_async_copy(pred_h.at[pl.ds(base, CROWS)], pb, sp).wait()
        pltpu.make_async_copy(lab_h.at[pl.ds(base, CROWS)], lb, sl).wait()

    def compute(pb, lb, acc):
        def row(r, acc):
            for v in range(VPR):
                x = pb[r, pl.ds(L * v, L)]
                lv = lb[r, pl.ds(L * v, L)]
                acc = acc + _elem(x, lv > 0, scv[v])
            return acc
        return lax.fori_loop(0, CROWS, row, acc)

    start(base, pb0, lb0, sp0, sl0)
    start(base + CROWS, pb1, lb1, sp1, sl1)

    def pair(g, acc):
        c0 = base + (2 * g) * CROWS
        wait(pb0, lb0, sp0, sl0)
        acc = compute(pb0, lb0, acc)
        start(c0 + 2 * CROWS, pb0, lb0, sp0, sl0)
        wait(pb1, lb1, sp1, sl1)
        acc = compute(pb1, lb1, acc)
        start(c0 + 3 * CROWS, pb1, lb1, sp1, sl1)
        return acc

    acc = lax.fori_loop(0, NCH // 2, pair, jnp.zeros((L,), jnp.float32))

    if NCH % 2:
        wait(pb0, lb0, sp0, sl0)
        acc = compute(pb0, lb0, acc)
        start(last, pb0, lb0, sp0, sl0)

    wait(pb0, lb0, sp0, sl0)
    wait(pb1, lb1, sp1, sl1)

    acc_v[...] = acc
    pltpu.sync_copy(acc_v, out_h.at[pl.ds(wid * L, L)])


def _sc_call(pred, label, score):
    mesh = plsc.VectorSubcoreMesh(core_axis_name="c", subcore_axis_name="s")
    f = functools.partial(
        pl.kernel,
        mesh=mesh,
        out_type=jax.ShapeDtypeStruct((NW * L,), jnp.float32),
        compiler_params=pltpu.CompilerParams(use_tc_tiling_on_sc=True),
        scratch_types=[
            pltpu.VMEM((CROWS, C), jnp.float32),
            pltpu.VMEM((CROWS, C), jnp.float32),
            pltpu.VMEM((CROWS, C), jnp.int32),
            pltpu.VMEM((CROWS, C), jnp.int32),
            pltpu.VMEM((C,), jnp.float32),
            pltpu.VMEM((L,), jnp.float32),
            pltpu.SemaphoreType.DMA,
            pltpu.SemaphoreType.DMA,
            pltpu.SemaphoreType.DMA,
            pltpu.SemaphoreType.DMA,
        ],
    )(_sc_body)
    return f(pred, label, score)


def _tc_body(score_ref, pred_ref, lab_ref, out_ref):
    x = pred_ref[...]
    pos = lab_ref[...] > 0
    sc = score_ref[...]
    ax = jnp.abs(x)
    u = jnp.exp(-ax)
    d = 1.0 / (1.0 + u)
    sp = jnp.maximum(x, 0.0) + jnp.log1p(u)
    s = jnp.where(x >= 0.0, d, 1.0 - d)
    scm = jnp.where(pos, sc, 0.0)
    a = jnp.where(pos, jnp.abs(sc - s), s)
    res = (sp - x * scm) * (a * lax.sqrt(a))

    @pl.when(pl.program_id(0) == 0)
    def _():
        out_ref[0, 0] = 0.0

    out_ref[0, 0] += jnp.sum(res)


def _tc_call(pred, label, score):
    return pl.pallas_call(
        _tc_body,
        grid=(G_TC,),
        in_specs=[
            pl.BlockSpec((1, C), lambda i: (0, 0)),
            pl.BlockSpec((BR, C), lambda i: (i, 0)),
            pl.BlockSpec((BR, C), lambda i: (i, 0)),
        ],
        out_specs=pl.BlockSpec((1, 1), lambda i: (0, 0),
                               memory_space=pltpu.SMEM),
        out_shape=jax.ShapeDtypeStruct((1, 1), jnp.float32),
    )(score.reshape(1, C), pred, label)


@jax.jit
def kernel(pred, label, score):
    sc_part = _sc_call(pred, label, score)
    tc_part = _tc_call(pred, label, score)
    return (jnp.sum(tc_part) + jnp.sum(sc_part)) / jnp.float32(TOTAL)
